# Initial kernel scaffold; baseline (speedup 1.0000x reference)
import jax
import jax.numpy as jnp
from jax.experimental import pallas as pl


def kernel(x, edge_index, W1, b1, W2, b2, tau, prev0, prev1):
    raise NotImplementedError("write your pallas kernel here")


# speedup vs baseline: 14.0416x; 14.0416x over previous
import functools

import jax
import jax.numpy as jnp
from jax import lax
from jax.experimental import pallas as pl
from jax.experimental.pallas import tpu as pltpu
from jax.experimental.pallas import tpu_sc as plsc

N = 10000
D = 128
E = 320000
NC = 2
NS = 16
NW = NC * NS
NPAD = 10112
TRASH = 10048
CH = 128
KC = 81
EPW = KC * CH
EPAD = NW * EPW
RPT = NPAD // NS
RB = 400
GRID = N // RB


def _deg_body(dst_hbm, zeros_hbm, ones_hbm, out_hbm, idx_v, ones_v, acc_sh, sem):
    del sem
    cid = lax.axis_index("c")
    sid = lax.axis_index("s")
    wid = cid * NS + sid

    pltpu.sync_copy(zeros_hbm.at[pl.ds(sid * RPT, RPT)],
                    acc_sh.at[pl.ds(sid * RPT, RPT)])

    pltpu.sync_copy(ones_hbm, ones_v)
    pltpu.sync_copy(dst_hbm.at[wid], idx_v)
    plsc.subcore_barrier()

    def body(j, _):
        pltpu.sync_copy(ones_v, acc_sh.at[idx_v.at[j]], add=True)
        return 0
    lax.fori_loop(0, KC, body, 0)

    plsc.subcore_barrier()
    pltpu.sync_copy(acc_sh.at[pl.ds(sid * RPT, RPT)],
                    out_hbm.at[cid, pl.ds(sid * RPT, RPT)])


def _seg_body(g_hbm, src_hbm, dst_hbm, zeros_hbm, out_hbm,
              sidx_v, didx_v, rows_v, acc_sh, sem):
    cid = lax.axis_index("c")
    sid = lax.axis_index("s")
    wid = cid * NS + sid

    pltpu.sync_copy(zeros_hbm.at[pl.ds(sid * RPT, RPT)],
                    acc_sh.at[pl.ds(sid * RPT, RPT)])
    pltpu.sync_copy(src_hbm.at[wid], sidx_v)
    pltpu.sync_copy(dst_hbm.at[wid], didx_v)
    plsc.subcore_barrier()

    def body(j, _):
        pltpu.async_copy(g_hbm.at[sidx_v.at[j]], rows_v, sem).wait()
        pltpu.sync_copy(rows_v, acc_sh.at[didx_v.at[j]], add=True)
        return 0
    lax.fori_loop(0, KC, body, 0)

    plsc.subcore_barrier()
    pltpu.sync_copy(acc_sh.at[pl.ds(sid * RPT, RPT)],
                    out_hbm.at[cid, pl.ds(sid * RPT, RPT)])


@functools.lru_cache(maxsize=None)
def _sc_kernels():
    mesh = plsc.VectorSubcoreMesh(core_axis_name="c", subcore_axis_name="s")
    deg = pl.kernel(
        _deg_body,
        out_type=jax.ShapeDtypeStruct((NC, NPAD, D), jnp.float32),
        mesh=mesh,
        scratch_types=[
            pltpu.VMEM((KC, CH), jnp.int32),
            pltpu.VMEM((CH, D), jnp.float32),
            pltpu.VMEM_SHARED((NPAD, D), jnp.float32),
            pltpu.SemaphoreType.DMA,
        ],
    )
    seg = pl.kernel(
        _seg_body,
        out_type=jax.ShapeDtypeStruct((NC, NPAD, D), jnp.float32),
        mesh=mesh,
        scratch_types=[
            pltpu.VMEM((KC, CH), jnp.int32),
            pltpu.VMEM((KC, CH), jnp.int32),
            pltpu.VMEM((CH, D), jnp.float32),
            pltpu.VMEM_SHARED((NPAD, D), jnp.float32),
            pltpu.SemaphoreType.DMA,
        ],
    )
    return deg, seg


def _dinv_block(degp_ref):
    deg = degp_ref[0, :, 0:1] + degp_ref[1, :, 0:1]
    return lax.rsqrt(jnp.maximum(deg, 1e-12))


def _mm1_body(x_ref, w_ref, degp_ref, o_ref):
    dinv = _dinv_block(degp_ref)
    o_ref[:, :] = jnp.dot(x_ref[:, :], w_ref[:, :],
                          preferred_element_type=jnp.float32) * dinv


def _comb_mm2_body(acc_ref, degp_ref, prev_ref, tau_ref, b_ref, w_ref,
                   emb_ref, g2_ref):
    dinv = _dinv_block(degp_ref)
    s = acc_ref[0] + acc_ref[1]
    h = jnp.maximum(s * dinv + b_ref[:, :], 0.0)
    tau = tau_ref[0, 0]
    emb = tau * prev_ref[:, :] + (1.0 - tau) * h
    emb_ref[:, :] = emb
    g2_ref[:, :] = jnp.dot(emb, w_ref[:, :],
                           preferred_element_type=jnp.float32) * dinv


def _comb_body(acc_ref, degp_ref, prev_ref, tau_ref, b_ref, emb_ref):
    dinv = _dinv_block(degp_ref)
    s = acc_ref[0] + acc_ref[1]
    h = jnp.maximum(s * dinv + b_ref[:, :], 0.0)
    tau = tau_ref[0, 0]
    emb_ref[:, :] = tau * prev_ref[:, :] + (1.0 - tau) * h


_spec_rows = pl.BlockSpec((RB, D), lambda i: (i, 0))
_spec_w = pl.BlockSpec((D, D), lambda i: (0, 0))
_spec_degp = pl.BlockSpec((NC, RB, D), lambda i: (0, i, 0))
_spec_acc = pl.BlockSpec((NC, RB, D), lambda i: (0, i, 0))
_spec_tau = pl.BlockSpec((1, 1), lambda i: (0, 0))
_spec_b = pl.BlockSpec((1, D), lambda i: (0, 0))
_shape_out = jax.ShapeDtypeStruct((N, D), jnp.float32)

_mm1 = pl.pallas_call(
    _mm1_body,
    grid=(GRID,),
    in_specs=[_spec_rows, _spec_w, _spec_degp],
    out_specs=_spec_rows,
    out_shape=_shape_out,
)

_comb_mm2 = pl.pallas_call(
    _comb_mm2_body,
    grid=(GRID,),
    in_specs=[_spec_acc, _spec_degp, _spec_rows, _spec_tau, _spec_b, _spec_w],
    out_specs=(_spec_rows, _spec_rows),
    out_shape=(_shape_out, _shape_out),
)

_comb = pl.pallas_call(
    _comb_body,
    grid=(GRID,),
    in_specs=[_spec_acc, _spec_degp, _spec_rows, _spec_tau, _spec_b],
    out_specs=_spec_rows,
    out_shape=_shape_out,
)


def kernel(x, edge_index, W1, b1, W2, b2, tau, prev0, prev1):
    deg_k, seg_k = _sc_kernels()

    loop = jnp.arange(N, dtype=jnp.int32)
    npad_e = EPAD - E - N
    src3 = jnp.concatenate(
        [edge_index[0], loop, jnp.zeros((npad_e,), jnp.int32)]
    ).reshape(NW, KC, CH)
    dst3 = jnp.concatenate(
        [edge_index[1], loop, jnp.full((npad_e,), TRASH, jnp.int32)]
    ).reshape(NW, KC, CH)

    ones_rows = jnp.ones((CH, D), jnp.float32)
    zerosD = jnp.zeros((NPAD, D), jnp.float32)
    tau2 = tau.reshape(1, 1)
    b1r = b1.reshape(1, D)
    b2r = b2.reshape(1, D)

    degp = deg_k(dst3, zerosD, ones_rows)
    g1 = _mm1(x, W1, degp)
    acc1 = seg_k(g1, src3, dst3, zerosD)
    emb0, g2 = _comb_mm2(acc1, degp, prev0, tau2, b1r, W2)
    acc2 = seg_k(g2, src3, dst3, zerosD)
    emb1 = _comb(acc2, degp, prev1, tau2, b2r)
    return (emb0, emb1)
